# baseline (speedup 1.0000x reference)
import jax
import jax.numpy as jnp
from jax import lax
from jax.experimental import pallas as pl
from jax.experimental.pallas import tpu as pltpu
from jax.experimental.pallas import tpu_sc as plsc

_PARENT = (1, 20, 20, 2, 20, 4, 5, 6, 20, 8, 9, 10, 0, 12, 13, 14, 0, 16,
           17, 18, 20, 22, 7, 24, 11)

_B, _C, _V, _T = 512, 3, 25, 300
_UNITS = _C * _T
_NW = 32
_Q, _R = divmod(_UNITS, _NW)
_NB = 3

_CHUNKS = _B // 16


def _compute(xbuf, obuf):
    @plsc.parallel_loop(0, _CHUNKS, 1)
    def do_chunk(k):
        off = pl.multiple_of(k * 16, 16)
        regs = [xbuf[j, pl.ds(off, 16)] for j in range(_V)]
        for j in range(_V):
            obuf[j, pl.ds(off, 16)] = regs[j] - regs[_PARENT[j]]


def _sc_body(x_hbm, out_hbm,
             xb0, xb1, xb2, ob0, ob1, ob2,
             si0, si1, si2, so0, so1, so2):
    wid = lax.axis_index("s") * 2 + lax.axis_index("c")
    base = wid * _Q + jnp.minimum(wid, _R)
    cnt = _Q + (wid < _R).astype(jnp.int32)
    xbufs, obufs = (xb0, xb1, xb2), (ob0, ob1, ob2)
    sins, souts = (si0, si1, si2), (so0, so1, so2)

    def src(i):
        u = base + i
        return x_hbm.at[u // _T, :, u % _T]

    def dst(i):
        u = base + i
        return out_hbm.at[u // _T, :, u % _T]

    pltpu.make_async_copy(src(0), xbufs[0], sins[0]).start()
    pltpu.make_async_copy(src(1), xbufs[1], sins[1]).start()

    def do_triple(gp, carry):
        for b in range(_NB):
            i = gp * _NB + b
            @pl.when(i + 2 < cnt)
            def _():
                pltpu.make_async_copy(
                    src(i + 2), xbufs[(b + 2) % _NB], sins[(b + 2) % _NB]
                ).start()

            @pl.when(i < cnt)
            def _():
                pltpu.make_async_copy(src(i), xbufs[b], sins[b]).wait()

            @pl.when(jnp.logical_and(i >= _NB, i < cnt))
            def _():
                pltpu.make_async_copy(obufs[b], dst(i - _NB), souts[b]).wait()

            @pl.when(i < cnt)
            def _():
                _compute(xbufs[b], obufs[b])
                pltpu.make_async_copy(obufs[b], dst(i), souts[b]).start()
        return carry

    lax.fori_loop(0, (_Q + 1 + _NB - 1) // _NB + 1, do_triple, 0)

    for b in range(_NB):
        pltpu.make_async_copy(obufs[b], dst(cnt - _NB + b), souts[b]).wait()


def kernel(joint_data):
    x = jnp.transpose(joint_data, (1, 2, 3, 0))
    mesh = plsc.VectorSubcoreMesh(core_axis_name="c", subcore_axis_name="s")
    f = pl.kernel(
        _sc_body,
        mesh=mesh,
        out_type=jax.ShapeDtypeStruct((_C, _V, _T, _B), jnp.float32),
        scratch_types=(
            [pltpu.VMEM((_V, _B), jnp.float32) for _ in range(2 * _NB)]
            + [pltpu.SemaphoreType.DMA for _ in range(2 * _NB)]
        ),
    )
    out = f(x)
    return jnp.transpose(out, (3, 0, 1, 2))
